# Initial kernel scaffold; baseline (speedup 1.0000x reference)
import jax
import jax.numpy as jnp
from jax.experimental import pallas as pl


def kernel(positions, node_attrs, shifts, params, edge_index, batch):
    raise NotImplementedError("write your pallas kernel here")


# speedup vs baseline: 6.2019x; 6.2019x over previous
import functools

import jax
import jax.numpy as jnp
import numpy as np
from jax import lax
from jax.experimental import pallas as pl
from jax.experimental.pallas import tpu as pltpu
from jax.experimental.pallas import tpu_sc as plsc

N = 10000
E = 320000
D = 128
DV = 32
NB = 8
G = 16
RMAX = 5.0
S3 = float(np.sqrt(3.0))

NPAD = 10240
EPAD = 327680
NC, NS = 2, 16
NW = NC * NS
CH = 128

EB = 2048
NBK = 1024


@functools.lru_cache(maxsize=None)
def _gather(Npad, K, Ep):
    e_w = Ep // NW
    nch = e_w // CH
    mesh = plsc.VectorSubcoreMesh(core_axis_name="c", subcore_axis_name="s")

    @functools.partial(
        pl.kernel, mesh=mesh,
        out_type=jax.ShapeDtypeStruct((Ep, K), jnp.float32),
        compiler_params=pltpu.CompilerParams(use_tc_tiling_on_sc=False),
        scratch_types=[
            pltpu.VMEM((e_w,), jnp.int32),
            pltpu.VMEM((2, CH, K), jnp.float32),
            pltpu.SemaphoreType.DMA,
        ],
    )
    def k(table_hbm, idx_hbm, out_hbm, idx_v, buf, sem):
        wid = lax.axis_index("s") * NC + lax.axis_index("c")
        base = wid * e_w
        pltpu.sync_copy(idx_hbm.at[pl.ds(base, e_w)], idx_v)

        def body(c, carry):
            pltpu.async_copy(
                table_hbm.at[idx_v.at[pl.ds(c * CH, CH)]], buf.at[0], sem
            ).wait()
            pltpu.sync_copy(buf.at[0], out_hbm.at[pl.ds(base + c * CH, CH)])
            return carry

        lax.fori_loop(0, nch, body, 0)

    return k


@functools.lru_cache(maxsize=None)
def _scatter_add(Ep, K, Npad):
    e_w = Ep // NS
    nch = e_w // CH
    Kh = K // 2
    rows_t = Npad // NS
    mesh = plsc.VectorSubcoreMesh(core_axis_name="c", subcore_axis_name="s")

    @functools.partial(
        pl.kernel, mesh=mesh,
        out_type=jax.ShapeDtypeStruct((Npad, K), jnp.float32),
        compiler_params=pltpu.CompilerParams(use_tc_tiling_on_sc=False),
        scratch_types=[
            pltpu.VMEM((CH,), jnp.int32),
            pltpu.VMEM((CH, Kh), jnp.float32),
            pltpu.VMEM_SHARED((Npad, Kh), jnp.float32),
        ],
    )
    def k(vals_hbm, idx3_hbm, zeros_hbm, out_hbm, idx_c, vbuf, acc):
        cid = lax.axis_index("c")
        sid = lax.axis_index("s")
        col0 = cid * Kh
        r0 = sid * rows_t
        pltpu.sync_copy(zeros_hbm.at[pl.ds(r0, rows_t), pl.ds(0, Kh)],
                        acc.at[pl.ds(r0, rows_t)])
        plsc.subcore_barrier()

        def body(c, carry):
            pltpu.sync_copy(idx3_hbm.at[sid, c], idx_c)
            pltpu.sync_copy(
                vals_hbm.at[pl.ds(sid * e_w + c * CH, CH), pl.ds(col0, Kh)],
                vbuf)
            pltpu.sync_copy(vbuf, acc.at[idx_c], add=True)
            return carry

        lax.fori_loop(0, nch, body, 0)
        plsc.subcore_barrier()
        pltpu.sync_copy(acc.at[pl.ds(r0, rows_t)],
                        out_hbm.at[pl.ds(r0, rows_t), pl.ds(col0, Kh)])

    return k


def _dot(a, b):
    return jnp.dot(a, b, preferred_element_type=jnp.float32)


def _sigmoid(x):
    return 1.0 / (1.0 + jnp.exp(-x))


def _silu(x):
    return x * _sigmoid(x)


def _dsilu(x):
    s = _sigmoid(x)
    return s * (1.0 + x * (1.0 - s))


def _edge_grid(n_in, n_out):
    eblk = EPAD // EB

    def specs(widths_in, widths_out):
        in_specs = [pl.BlockSpec((EB, w), lambda i: (i, 0)) for w in widths_in]
        out_specs = [pl.BlockSpec((EB, w), lambda i: (i, 0)) for w in widths_out]
        return eblk, in_specs, out_specs

    return specs


def _wspec(shape):
    return pl.BlockSpec(shape, lambda i: tuple(0 for _ in shape))


def _run_edge(body, ins, widths_in, wparams, widths_out):
    eblk = EPAD // EB
    in_specs = ([pl.BlockSpec((EB, w), lambda i: (i, 0)) for w in widths_in]
                + [_wspec(w.shape) for w in wparams])
    out_specs = [pl.BlockSpec((EB, w), lambda i: (i, 0)) for w in widths_out]
    out_shape = [jax.ShapeDtypeStruct((EPAD, w), jnp.float32) for w in widths_out]
    outs = pl.pallas_call(
        body, grid=(eblk,), in_specs=in_specs, out_specs=out_specs,
        out_shape=out_shape,
    )(*ins, *wparams)
    return outs


def _run_node(body, ins, widths_in, wparams, widths_out):
    nblk = NPAD // NBK
    in_specs = ([pl.BlockSpec((NBK, w), lambda i: (i, 0)) for w in widths_in]
                + [_wspec(w.shape) for w in wparams])
    out_specs = [pl.BlockSpec((NBK, w), lambda i: (i, 0)) for w in widths_out]
    out_shape = [jax.ShapeDtypeStruct((NPAD, w), jnp.float32) for w in widths_out]
    outs = pl.pallas_call(
        body, grid=(nblk,), in_specs=in_specs, out_specs=out_specs,
        out_shape=out_shape,
    )(*ins, *wparams)
    return outs


def _node_pre_body(na_ref, wemb_ref, wups0_ref, wes0_ref, wes1_ref, ae_ref,
                   s0_ref, s0up_ref, es0_ref, es1_ref, ne0_ref):
    na = na_ref[...]
    s0 = _dot(na, wemb_ref[...])
    s0_ref[...] = s0
    s0up_ref[...] = _dot(s0, wups0_ref[...])
    es0_ref[...] = _dot(na, wes0_ref[...])
    es1_ref[...] = _dot(na, wes1_ref[...])
    ne0_ref[...] = _dot(na, ae_ref[...])


def _geom_body(ps_ref, pr_ref, sh_ref, geom_ref):
    vec = pr_ref[...][:, 0:3] - ps_ref[...][:, 0:3] + sh_ref[...][:, 0:3]
    r = jnp.sqrt(jnp.sum(vec * vec, axis=1, keepdims=True) + 1e-12)
    u = vec / r
    sh1 = S3 * u
    n = (lax.broadcasted_iota(jnp.int32, (EB, NB), 1) + 1).astype(jnp.float32)
    kn = n * (np.pi / RMAX)
    S = np.sqrt(2.0 / RMAX)
    arg = kn * r
    sn, cs = jnp.sin(arg), jnp.cos(arg)
    b = S * sn / r
    bp = S * (kn * cs / r - sn / (r * r))
    ur = r * (1.0 / RMAX)
    u2 = ur * ur
    u4 = u2 * u2
    u5 = u4 * ur
    env = 1.0 + u5 * (ur * (-28.0 + 48.0 * ur - 21.0 * u2))
    envp = u4 * ur * (-168.0 + 336.0 * ur - 168.0 * u2) * (1.0 / RMAX)
    inside = ur < 1.0
    c = jnp.where(inside, env, 0.0)
    cp = jnp.where(inside, envp, 0.0)
    ef = b * c
    efdr = bp * c + b * cp
    pad = jnp.zeros((EB, 12), jnp.float32)
    geom_ref[...] = jnp.concatenate([ef, efdr, sh1, r, pad], axis=1)


def _edge_fwd0_body(geom_ref, gj_ref, wr1_ref, wr2_ref, psv_ref, msg_ref):
    g = geom_ref[...]
    ef = g[:, 0:NB]
    h = _dot(ef, wr1_ref[...])
    Rm = _dot(_silu(h), wr2_ref[...])
    R0 = Rm[:, 0:D]
    R2 = Rm[:, D + DV:D + 2 * DV]
    sj = gj_ref[...]
    m_s = R0 * sj
    q = R2 * _dot(sj, psv_ref[...])
    mv = [q * g[:, 16 + d:17 + d] for d in range(3)]
    msg_ref[...] = jnp.concatenate([m_s] + mv, axis=1)


def _node0_body(smsg_ref, s0_ref, es0_ref, wouts0_ref, wskips0_ref,
                woutv0_ref, wread0_ref, wups1_ref, wupv1_ref,
                s1_ref, v1_ref, gj1_ref, en0_ref):
    sm = smsg_ref[...]
    s1 = _dot(sm[:, 0:D], wouts0_ref[...]) + _dot(s0_ref[...], wskips0_ref[...]) * es0_ref[...]
    v1 = [_dot(sm[:, D + DV * d:D + DV * (d + 1)], woutv0_ref[...]) for d in range(3)]
    s1_ref[...] = s1
    v1_ref[...] = jnp.concatenate(v1, axis=1)
    en0_ref[...] = _dot(s1, wread0_ref[...])
    s1up = _dot(s1, wups1_ref[...])
    v1up = [_dot(v1[d], wupv1_ref[...]) for d in range(3)]
    gj1_ref[...] = jnp.concatenate([s1up] + v1up, axis=1)


def _edge_fwd1_body(geom_ref, gj_ref, wr1_ref, wr2_ref, pvs_ref, psv_ref,
                    msg_ref):
    g = geom_ref[...]
    ef = g[:, 0:NB]
    h = _dot(ef, wr1_ref[...])
    Rm = _dot(_silu(h), wr2_ref[...])
    R0 = Rm[:, 0:D]
    R1 = Rm[:, D:D + DV]
    R2 = Rm[:, D + DV:D + 2 * DV]
    R3 = Rm[:, D + 2 * DV:D + 3 * DV]
    gj = gj_ref[...]
    sj = gj[:, 0:D]
    vj = [gj[:, D + DV * d:D + DV * (d + 1)] for d in range(3)]
    vdot = sum(vj[d] * g[:, 16 + d:17 + d] for d in range(3))
    m_s = R0 * sj + _dot(R1 * vdot, pvs_ref[...])
    q = R2 * _dot(sj, psv_ref[...])
    mv = [q * g[:, 16 + d:17 + d] + R3 * vj[d] for d in range(3)]
    msg_ref[...] = jnp.concatenate([m_s] + mv, axis=1)


def _node1_body(smsg_ref, s1_ref, es1_ref, wouts1_ref, wskips1_ref,
                wread1_ref, wread2r_ref, wread1t_ref, wouts1t_ref,
                wskips1t_ref, wread0r_ref,
                en1_ref, as1_ref, ds1p_ref):
    sm = smsg_ref[...]
    es1 = es1_ref[...]
    s2 = _dot(sm[:, 0:D], wouts1_ref[...]) + _dot(s1_ref[...], wskips1_ref[...]) * es1
    t = _dot(s2, wread1_ref[...])
    sg = _sigmoid(t)
    en1_ref[...] = _dot(t * sg, wread2r_ref[...])
    gt = (sg * (1.0 + t * (1.0 - sg))) * wread2r_ref[...][:, 0][None, :]
    g_s2 = _dot(gt, wread1t_ref[...])
    as1_ref[...] = _dot(g_s2, wouts1t_ref[...])
    ds1p_ref[...] = _dot(g_s2 * es1, wskips1t_ref[...]) + wread0r_ref[...]


def _edge_bwd1_body(geom_ref, gj_ref, dmg_ref, wr1_ref, wr2ab_ref, pvst_ref,
                    w2at_ref, w2bt_ref, wr1t_ref, dgj_ref, dvec_ref):
    g = geom_ref[...]
    ef = g[:, 0:NB]
    efdr = g[:, NB:2 * NB]
    sh = g[:, 16:19]
    r = g[:, 19:20]
    h = _dot(ef, wr1_ref[...])
    a = _silu(h)
    Rab = _dot(a, wr2ab_ref[...])
    R0 = Rab[:, 0:D]
    R1 = Rab[:, D:D + DV]
    gj = gj_ref[...]
    sj = gj[:, 0:D]
    vj = [gj[:, D + DV * d:D + DV * (d + 1)] for d in range(3)]
    vdot = sum(vj[d] * sh[:, d:d + 1] for d in range(3))
    dm_s = dmg_ref[...]
    dsj = dm_s * R0
    dRV = _dot(dm_s, pvst_ref[...])
    dR1 = dRV * vdot
    dvdot = dRV * R1
    dvj = [dvdot * sh[:, d:d + 1] for d in range(3)]
    dsh = jnp.concatenate(
        [jnp.sum(dvdot * vj[d], axis=1, keepdims=True) for d in range(3)],
        axis=1)
    dR0 = dm_s * sj
    da = _dot(dR0, w2at_ref[...]) + _dot(dR1, w2bt_ref[...])
    defr = _dot(da * _dsilu(h), wr1t_ref[...])
    u = sh * (1.0 / S3)
    ddot = jnp.sum(defr * efdr, axis=1, keepdims=True)
    udsh = jnp.sum(dsh * u, axis=1, keepdims=True)
    dvec = u * ddot + (S3 / r) * (dsh - u * udsh)
    dgj_ref[...] = jnp.concatenate([dsj] + dvj, axis=1)
    dvec_ref[...] = jnp.concatenate(
        [dvec, jnp.zeros((EB, 29), jnp.float32)], axis=1)


def _node_bwd1_body(dsup_ref, ds1p_ref, wups1t_ref, wupv1t_ref, wouts0t_ref,
                    woutv0t_ref, a0_ref):
    dsup = dsup_ref[...]
    ds1 = ds1p_ref[...] + _dot(dsup[:, 0:D], wups1t_ref[...])
    dv1 = [_dot(dsup[:, D + DV * d:D + DV * (d + 1)], wupv1t_ref[...])
           for d in range(3)]
    a_s0 = _dot(ds1, wouts0t_ref[...])
    a_v0 = [_dot(dv1[d], woutv0t_ref[...]) for d in range(3)]
    a0_ref[...] = jnp.concatenate([a_s0] + a_v0, axis=1)


def _edge_bwd0_body(geom_ref, gj_ref, dmg_ref, dvec1_ref, wr1_ref, wr2c_ref,
                    psv_ref, w2at_ref, w2ct_ref, wr1t_ref,
                    dvp_ref, dvn_ref):
    g = geom_ref[...]
    ef = g[:, 0:NB]
    efdr = g[:, NB:2 * NB]
    sh = g[:, 16:19]
    r = g[:, 19:20]
    h = _dot(ef, wr1_ref[...])
    a = _silu(h)
    R2 = _dot(a, wr2c_ref[...])
    sj = gj_ref[...]
    sjv = _dot(sj, psv_ref[...])
    q = R2 * sjv
    dmg = dmg_ref[...]
    dm_s = dmg[:, 0:D]
    dm_v = [dmg[:, D + DV * d:D + DV * (d + 1)] for d in range(3)]
    dR0 = dm_s * sj
    dQ = sum(dm_v[d] * sh[:, d:d + 1] for d in range(3))
    dR2 = dQ * sjv
    dsh = jnp.concatenate(
        [jnp.sum(dm_v[d] * q, axis=1, keepdims=True) for d in range(3)],
        axis=1)
    da = _dot(dR0, w2at_ref[...]) + _dot(dR2, w2ct_ref[...])
    defr = _dot(da * _dsilu(h), wr1t_ref[...])
    u = sh * (1.0 / S3)
    ddot = jnp.sum(defr * efdr, axis=1, keepdims=True)
    udsh = jnp.sum(dsh * u, axis=1, keepdims=True)
    dvec = u * ddot + (S3 / r) * (dsh - u * udsh) + dvec1_ref[...][:, 0:3]
    z = jnp.zeros((EB, 29), jnp.float32)
    dvp_ref[...] = jnp.concatenate([dvec, z], axis=1)
    dvn_ref[...] = jnp.concatenate([-dvec, z], axis=1)


def _segsum_body(b_ref, ne0_ref, en0_ref, en1_ref, out_ref):
    i = pl.program_id(0)

    @pl.when(i == 0)
    def _():
        out_ref[...] = jnp.zeros((8, G), jnp.float32)

    b = b_ref[...][0, 0, :]
    onehot = (b[:, None] == lax.broadcasted_iota(jnp.int32, (NBK, G), 1))
    onehot = onehot.astype(jnp.float32)
    for j, ref in enumerate((ne0_ref, en0_ref, en1_ref)):
        val = ref[...][:, 0:1]
        out_ref[j:j + 1, :] += jnp.sum(val * onehot, axis=0, keepdims=True)


def kernel(positions, node_attrs, shifts, params, edge_index, batch):
    p0, p1 = params['interactions'][0], params['interactions'][1]
    f32 = jnp.float32

    snd = edge_index[0].astype(jnp.int32)
    rcv = edge_index[1].astype(jnp.int32)
    snd_g = jnp.pad(snd, (0, EPAD - E))
    rcv_g = jnp.pad(rcv, (0, EPAD - E))
    snd_s3 = jnp.pad(snd, (0, EPAD - E), constant_values=N).reshape(NS, -1, CH)
    rcv_s3 = jnp.pad(rcv, (0, EPAD - E), constant_values=N).reshape(NS, -1, CH)
    fidx = jnp.concatenate(
        [jnp.pad(snd, (0, EPAD - E), constant_values=N),
         jnp.pad(rcv, (0, EPAD - E), constant_values=N)]).reshape(NS, -1, CH)

    pos_t = jnp.zeros((NPAD, 16), f32).at[:N, 0:3].set(positions)
    na_t = jnp.zeros((NPAD, 16), f32).at[:N, :10].set(node_attrs)
    shifts_t = jnp.zeros((EPAD, 16), f32).at[:E, 0:3].set(shifts)
    batch3 = jnp.pad(batch.astype(jnp.int32), (0, NPAD - N),
                     constant_values=G).reshape(NPAD // NBK, 1, NBK)
    z224 = jnp.zeros((NPAD, 224), f32)
    z32 = jnp.zeros((NPAD, 32), f32)

    def padw(w, rows=16):
        out = jnp.zeros((rows, w.shape[1]), f32)
        return out.at[:w.shape[0]].set(w)

    wread0_8 = jnp.pad(p0['w_read'], ((0, 0), (0, 7)))
    wread2_8 = jnp.pad(p1['wread2'], ((0, 0), (0, 7)))
    ae_8 = jnp.pad(params['atomic_energies'][:, None], ((0, 6), (0, 7)))

    s0, s0up, es0, es1, ne0 = _run_node(
        _node_pre_body, [na_t], [16],
        [padw(params['W_embed']), p0['Wup_s'], padw(p0['w_elem_s']),
         padw(p1['w_elem_s']), ae_8],
        [D, D, D, D, 8])

    psr = _gather(NPAD, 16, 2 * EPAD)(pos_t, jnp.concatenate([snd_g, rcv_g]))
    ps, pr = psr[:EPAD], psr[EPAD:]
    geom, = _run_edge(_geom_body, [ps, pr, shifts_t], [16, 16, 16], [], [32])

    sj0 = _gather(NPAD, D, EPAD)(s0up + psr[0, 0] * 0.0, snd_g)
    msg0, = _run_edge(_edge_fwd0_body, [geom, sj0], [32, D],
                      [p0['Wr1'], p0['Wr2'], p0['P_sv']], [224])
    smsg0 = _scatter_add(EPAD, 224, NPAD)(msg0, rcv_s3, z224)
    s1, v1, gj1t, en0 = _run_node(
        _node0_body, [smsg0, s0, es0], [224, D, D],
        [p0['Wout_s'], p0['Wskip_s'], p0['Wout_v'], wread0_8,
         p1['Wup_s'], p1['Wup_v']],
        [D, 96, 224, 8])

    gj1 = _gather(NPAD, 224, EPAD)(gj1t, snd_g)
    msg1, = _run_edge(_edge_fwd1_body, [geom, gj1], [32, 224],
                      [p1['Wr1'], p1['Wr2'], p1['P_vs'], p1['P_sv']], [224])
    smsg1 = _scatter_add(EPAD, 224, NPAD)(msg1, rcv_s3, z224)
    en1, as1, ds1p = _run_node(
        _node1_body, [smsg1, s1, es1], [224, D, D],
        [p1['Wout_s'], p1['Wskip_s'], p1['Wread1'], wread2_8,
         p1['Wread1'].T, p1['Wout_s'].T, p1['Wskip_s'].T,
         jnp.broadcast_to(p0['w_read'][:, 0][None, :], (1, D))],
        [8, D, D])

    dmg1 = _gather(NPAD, D, EPAD)(as1, rcv_g)
    dgj1, dvec1 = _run_edge(
        _edge_bwd1_body, [geom, gj1, dmg1], [32, 224, D],
        [p1['Wr1'], p1['Wr2'][:, :D + DV], p1['P_vs'].T,
         p1['Wr2'][:, :D].T, p1['Wr2'][:, D:D + DV].T, p1['Wr1'].T],
        [224, 32])
    dsup1 = _scatter_add(EPAD, 224, NPAD)(dgj1, snd_s3, z224)
    a0, = _run_node(
        _node_bwd1_body, [dsup1, ds1p], [224, D],
        [p1['Wup_s'].T, p1['Wup_v'].T, p0['Wout_s'].T, p0['Wout_v'].T],
        [224])

    dmg0 = _gather(NPAD, 224, EPAD)(a0, rcv_g)
    dvp, dvn = _run_edge(
        _edge_bwd0_body, [geom, sj0, dmg0, dvec1], [32, D, 224, 32],
        [p0['Wr1'], p0['Wr2'][:, D + DV:D + 2 * DV], p0['P_sv'],
         p0['Wr2'][:, :D].T, p0['Wr2'][:, D + DV:D + 2 * DV].T, p0['Wr1'].T],
        [32, 32])

    fvals = jnp.concatenate([dvp, dvn], axis=0)
    ftab = _scatter_add(2 * EPAD, 32, NPAD)(fvals, fidx, z32)
    forces = ftab[:N, 0:3]

    esum = pl.pallas_call(
        _segsum_body, grid=(NPAD // NBK,),
        in_specs=[pl.BlockSpec((1, 1, NBK), lambda i: (i, 0, 0)),
                  pl.BlockSpec((NBK, 8), lambda i: (i, 0)),
                  pl.BlockSpec((NBK, 8), lambda i: (i, 0)),
                  pl.BlockSpec((NBK, 8), lambda i: (i, 0))],
        out_specs=pl.BlockSpec((8, G), lambda i: (0, 0)),
        out_shape=jax.ShapeDtypeStruct((8, G), jnp.float32),
    )(batch3, ne0, en0, en1)
    contributions = esum[0:3, :].T
    total = jnp.sum(contributions, axis=-1)
    return total, contributions, forces


# speedup vs baseline: 6.8819x; 1.1096x over previous
import functools

import jax
import jax.numpy as jnp
import numpy as np
from jax import lax
from jax.experimental import pallas as pl
from jax.experimental.pallas import tpu as pltpu
from jax.experimental.pallas import tpu_sc as plsc

N = 10000
E = 320000
D = 128
DV = 32
NB = 8
G = 16
RMAX = 5.0
S3 = float(np.sqrt(3.0))

NPAD = 10240
EPAD = 327680
NC, NS = 2, 16
NW = NC * NS
CH = 128

EB = 2048
NBK = 1024


@functools.lru_cache(maxsize=None)
def _gather(Npad, K, Ep):
    e_w = Ep // NW
    nch = e_w // CH
    mesh = plsc.VectorSubcoreMesh(core_axis_name="c", subcore_axis_name="s")

    assert nch % 2 == 0

    @functools.partial(
        pl.kernel, mesh=mesh,
        out_type=jax.ShapeDtypeStruct((Ep, K), jnp.float32),
        compiler_params=pltpu.CompilerParams(use_tc_tiling_on_sc=False),
        scratch_types=[
            pltpu.VMEM((e_w,), jnp.int32),
            pltpu.VMEM((2, CH, K), jnp.float32),
            pltpu.SemaphoreType.DMA,
            pltpu.SemaphoreType.DMA,
        ],
    )
    def k(table_hbm, idx_hbm, out_hbm, idx_v, buf, semA, semB):
        wid = lax.axis_index("s") * NC + lax.axis_index("c")
        base = wid * e_w

        def issue(c, slot, sem):
            pltpu.async_copy(
                table_hbm.at[idx_v.at[pl.ds(c * CH, CH)]], buf.at[slot], sem)

        def drain(c, slot, sem):
            pltpu.make_async_copy(
                table_hbm.at[idx_v.at[pl.ds(c * CH, CH)]], buf.at[slot], sem
            ).wait()
            pltpu.sync_copy(buf.at[slot], out_hbm.at[pl.ds(base + c * CH, CH)])

        pltpu.sync_copy(idx_hbm.at[pl.ds(base, e_w)], idx_v)
        issue(0, 0, semA)

        def body(g, carry):
            c0 = 2 * g
            issue(c0 + 1, 1, semB)
            drain(c0, 0, semA)

            @pl.when(c0 + 2 < nch)
            def _():
                issue(c0 + 2, 0, semA)

            drain(c0 + 1, 1, semB)
            return carry

        lax.fori_loop(0, nch // 2, body, 0)

    return k


@functools.lru_cache(maxsize=None)
def _scatter_add(Ep, K, Npad):
    e_w = Ep // NS
    nch = e_w // CH
    Kh = K // 2
    rows_t = Npad // NS
    mesh = plsc.VectorSubcoreMesh(core_axis_name="c", subcore_axis_name="s")

    assert nch % 2 == 0

    @functools.partial(
        pl.kernel, mesh=mesh,
        out_type=jax.ShapeDtypeStruct((Npad, K), jnp.float32),
        compiler_params=pltpu.CompilerParams(use_tc_tiling_on_sc=False),
        scratch_types=[
            pltpu.VMEM((2, CH), jnp.int32),
            pltpu.VMEM((2, CH, Kh), jnp.float32),
            pltpu.VMEM_SHARED((Npad, Kh), jnp.float32),
            pltpu.SemaphoreType.DMA,
            pltpu.SemaphoreType.DMA,
        ],
    )
    def k(vals_hbm, idx3_hbm, zeros_hbm, out_hbm, idx_c, vbuf, acc,
          semA, semB):
        cid = lax.axis_index("c")
        sid = lax.axis_index("s")
        col0 = cid * Kh
        r0 = sid * rows_t
        pltpu.sync_copy(zeros_hbm.at[pl.ds(r0, rows_t), pl.ds(0, Kh)],
                        acc.at[pl.ds(r0, rows_t)])
        plsc.subcore_barrier()

        def issue(c, slot, sem):
            pltpu.async_copy(idx3_hbm.at[sid, c], idx_c.at[slot], sem)
            pltpu.async_copy(
                vals_hbm.at[pl.ds(sid * e_w + c * CH, CH), pl.ds(col0, Kh)],
                vbuf.at[slot], sem)

        def drain_add(c, slot, sem):
            pltpu.make_async_copy(idx3_hbm.at[sid, c], idx_c.at[slot],
                                  sem).wait()
            pltpu.make_async_copy(
                vals_hbm.at[pl.ds(sid * e_w + c * CH, CH), pl.ds(col0, Kh)],
                vbuf.at[slot], sem).wait()
            pltpu.sync_copy(vbuf.at[slot], acc.at[idx_c.at[slot]], add=True)

        issue(0, 0, semA)

        def body(g, carry):
            c0 = 2 * g
            issue(c0 + 1, 1, semB)
            drain_add(c0, 0, semA)

            @pl.when(c0 + 2 < nch)
            def _():
                issue(c0 + 2, 0, semA)

            drain_add(c0 + 1, 1, semB)
            return carry

        lax.fori_loop(0, nch // 2, body, 0)
        plsc.subcore_barrier()
        pltpu.sync_copy(acc.at[pl.ds(r0, rows_t)],
                        out_hbm.at[pl.ds(r0, rows_t), pl.ds(col0, Kh)])

    return k


def _dot(a, b):
    return jnp.dot(a, b, preferred_element_type=jnp.float32)


def _sigmoid(x):
    return 1.0 / (1.0 + jnp.exp(-x))


def _silu(x):
    return x * _sigmoid(x)


def _dsilu(x):
    s = _sigmoid(x)
    return s * (1.0 + x * (1.0 - s))


def _edge_grid(n_in, n_out):
    eblk = EPAD // EB

    def specs(widths_in, widths_out):
        in_specs = [pl.BlockSpec((EB, w), lambda i: (i, 0)) for w in widths_in]
        out_specs = [pl.BlockSpec((EB, w), lambda i: (i, 0)) for w in widths_out]
        return eblk, in_specs, out_specs

    return specs


def _wspec(shape):
    return pl.BlockSpec(shape, lambda i: tuple(0 for _ in shape))


def _run_edge(body, ins, widths_in, wparams, widths_out):
    eblk = EPAD // EB
    in_specs = ([pl.BlockSpec((EB, w), lambda i: (i, 0)) for w in widths_in]
                + [_wspec(w.shape) for w in wparams])
    out_specs = [pl.BlockSpec((EB, w), lambda i: (i, 0)) for w in widths_out]
    out_shape = [jax.ShapeDtypeStruct((EPAD, w), jnp.float32) for w in widths_out]
    outs = pl.pallas_call(
        body, grid=(eblk,), in_specs=in_specs, out_specs=out_specs,
        out_shape=out_shape,
    )(*ins, *wparams)
    return outs


def _run_node(body, ins, widths_in, wparams, widths_out):
    nblk = NPAD // NBK
    in_specs = ([pl.BlockSpec((NBK, w), lambda i: (i, 0)) for w in widths_in]
                + [_wspec(w.shape) for w in wparams])
    out_specs = [pl.BlockSpec((NBK, w), lambda i: (i, 0)) for w in widths_out]
    out_shape = [jax.ShapeDtypeStruct((NPAD, w), jnp.float32) for w in widths_out]
    outs = pl.pallas_call(
        body, grid=(nblk,), in_specs=in_specs, out_specs=out_specs,
        out_shape=out_shape,
    )(*ins, *wparams)
    return outs


def _node_pre_body(na_ref, wemb_ref, wups0_ref, wes0_ref, wes1_ref, ae_ref,
                   s0_ref, s0up_ref, es0_ref, es1_ref, ne0_ref):
    na = na_ref[...]
    s0 = _dot(na, wemb_ref[...])
    s0_ref[...] = s0
    s0up_ref[...] = _dot(s0, wups0_ref[...])
    es0_ref[...] = _dot(na, wes0_ref[...])
    es1_ref[...] = _dot(na, wes1_ref[...])
    ne0_ref[...] = _dot(na, ae_ref[...])


def _geom_body(ps_ref, pr_ref, sh_ref, geom_ref):
    vec = pr_ref[...][:, 0:3] - ps_ref[...][:, 0:3] + sh_ref[...][:, 0:3]
    r = jnp.sqrt(jnp.sum(vec * vec, axis=1, keepdims=True) + 1e-12)
    u = vec / r
    sh1 = S3 * u
    n = (lax.broadcasted_iota(jnp.int32, (EB, NB), 1) + 1).astype(jnp.float32)
    kn = n * (np.pi / RMAX)
    S = np.sqrt(2.0 / RMAX)
    arg = kn * r
    sn, cs = jnp.sin(arg), jnp.cos(arg)
    b = S * sn / r
    bp = S * (kn * cs / r - sn / (r * r))
    ur = r * (1.0 / RMAX)
    u2 = ur * ur
    u4 = u2 * u2
    u5 = u4 * ur
    env = 1.0 + u5 * (ur * (-28.0 + 48.0 * ur - 21.0 * u2))
    envp = u4 * ur * (-168.0 + 336.0 * ur - 168.0 * u2) * (1.0 / RMAX)
    inside = ur < 1.0
    c = jnp.where(inside, env, 0.0)
    cp = jnp.where(inside, envp, 0.0)
    ef = b * c
    efdr = bp * c + b * cp
    pad = jnp.zeros((EB, 12), jnp.float32)
    geom_ref[...] = jnp.concatenate([ef, efdr, sh1, r, pad], axis=1)


def _edge_fwd0_body(geom_ref, gj_ref, wr1_ref, wr2_ref, psv_ref, msg_ref):
    g = geom_ref[...]
    ef = g[:, 0:NB]
    h = _dot(ef, wr1_ref[...])
    Rm = _dot(_silu(h), wr2_ref[...])
    R0 = Rm[:, 0:D]
    R2 = Rm[:, D + DV:D + 2 * DV]
    sj = gj_ref[...]
    m_s = R0 * sj
    q = R2 * _dot(sj, psv_ref[...])
    mv = [q * g[:, 16 + d:17 + d] for d in range(3)]
    msg_ref[...] = jnp.concatenate([m_s] + mv, axis=1)


def _node0_body(smsg_ref, s0_ref, es0_ref, wouts0_ref, wskips0_ref,
                woutv0_ref, wread0_ref, wups1_ref, wupv1_ref,
                s1_ref, v1_ref, gj1_ref, en0_ref):
    sm = smsg_ref[...]
    s1 = _dot(sm[:, 0:D], wouts0_ref[...]) + _dot(s0_ref[...], wskips0_ref[...]) * es0_ref[...]
    v1 = [_dot(sm[:, D + DV * d:D + DV * (d + 1)], woutv0_ref[...]) for d in range(3)]
    s1_ref[...] = s1
    v1_ref[...] = jnp.concatenate(v1, axis=1)
    en0_ref[...] = _dot(s1, wread0_ref[...])
    s1up = _dot(s1, wups1_ref[...])
    v1up = [_dot(v1[d], wupv1_ref[...]) for d in range(3)]
    gj1_ref[...] = jnp.concatenate([s1up] + v1up, axis=1)


def _edge_fwd1_body(geom_ref, gj_ref, wr1_ref, wr2_ref, pvs_ref, psv_ref,
                    msg_ref):
    g = geom_ref[...]
    ef = g[:, 0:NB]
    h = _dot(ef, wr1_ref[...])
    Rm = _dot(_silu(h), wr2_ref[...])
    R0 = Rm[:, 0:D]
    R1 = Rm[:, D:D + DV]
    R2 = Rm[:, D + DV:D + 2 * DV]
    R3 = Rm[:, D + 2 * DV:D + 3 * DV]
    gj = gj_ref[...]
    sj = gj[:, 0:D]
    vj = [gj[:, D + DV * d:D + DV * (d + 1)] for d in range(3)]
    vdot = sum(vj[d] * g[:, 16 + d:17 + d] for d in range(3))
    m_s = R0 * sj + _dot(R1 * vdot, pvs_ref[...])
    q = R2 * _dot(sj, psv_ref[...])
    mv = [q * g[:, 16 + d:17 + d] + R3 * vj[d] for d in range(3)]
    msg_ref[...] = jnp.concatenate([m_s] + mv, axis=1)


def _node1_body(smsg_ref, s1_ref, es1_ref, wouts1_ref, wskips1_ref,
                wread1_ref, wread2r_ref, wread1t_ref, wouts1t_ref,
                wskips1t_ref, wread0r_ref,
                en1_ref, as1_ref, ds1p_ref):
    sm = smsg_ref[...]
    es1 = es1_ref[...]
    s2 = _dot(sm[:, 0:D], wouts1_ref[...]) + _dot(s1_ref[...], wskips1_ref[...]) * es1
    t = _dot(s2, wread1_ref[...])
    sg = _sigmoid(t)
    en1_ref[...] = _dot(t * sg, wread2r_ref[...])
    gt = (sg * (1.0 + t * (1.0 - sg))) * wread2r_ref[...][:, 0][None, :]
    g_s2 = _dot(gt, wread1t_ref[...])
    as1_ref[...] = _dot(g_s2, wouts1t_ref[...])
    ds1p_ref[...] = _dot(g_s2 * es1, wskips1t_ref[...]) + wread0r_ref[...]


def _edge_bwd1_body(geom_ref, gj_ref, dmg_ref, wr1_ref, wr2ab_ref, pvst_ref,
                    w2at_ref, w2bt_ref, wr1t_ref, dgj_ref, dvec_ref):
    g = geom_ref[...]
    ef = g[:, 0:NB]
    efdr = g[:, NB:2 * NB]
    sh = g[:, 16:19]
    r = g[:, 19:20]
    h = _dot(ef, wr1_ref[...])
    a = _silu(h)
    Rab = _dot(a, wr2ab_ref[...])
    R0 = Rab[:, 0:D]
    R1 = Rab[:, D:D + DV]
    gj = gj_ref[...]
    sj = gj[:, 0:D]
    vj = [gj[:, D + DV * d:D + DV * (d + 1)] for d in range(3)]
    vdot = sum(vj[d] * sh[:, d:d + 1] for d in range(3))
    dm_s = dmg_ref[...]
    dsj = dm_s * R0
    dRV = _dot(dm_s, pvst_ref[...])
    dR1 = dRV * vdot
    dvdot = dRV * R1
    dvj = [dvdot * sh[:, d:d + 1] for d in range(3)]
    dsh = jnp.concatenate(
        [jnp.sum(dvdot * vj[d], axis=1, keepdims=True) for d in range(3)],
        axis=1)
    dR0 = dm_s * sj
    da = _dot(dR0, w2at_ref[...]) + _dot(dR1, w2bt_ref[...])
    defr = _dot(da * _dsilu(h), wr1t_ref[...])
    u = sh * (1.0 / S3)
    ddot = jnp.sum(defr * efdr, axis=1, keepdims=True)
    udsh = jnp.sum(dsh * u, axis=1, keepdims=True)
    dvec = u * ddot + (S3 / r) * (dsh - u * udsh)
    dgj_ref[...] = jnp.concatenate([dsj] + dvj, axis=1)
    dvec_ref[...] = jnp.concatenate(
        [dvec, jnp.zeros((EB, 29), jnp.float32)], axis=1)


def _node_bwd1_body(dsup_ref, ds1p_ref, wups1t_ref, wupv1t_ref, wouts0t_ref,
                    woutv0t_ref, a0_ref):
    dsup = dsup_ref[...]
    ds1 = ds1p_ref[...] + _dot(dsup[:, 0:D], wups1t_ref[...])
    dv1 = [_dot(dsup[:, D + DV * d:D + DV * (d + 1)], wupv1t_ref[...])
           for d in range(3)]
    a_s0 = _dot(ds1, wouts0t_ref[...])
    a_v0 = [_dot(dv1[d], woutv0t_ref[...]) for d in range(3)]
    a0_ref[...] = jnp.concatenate([a_s0] + a_v0, axis=1)


def _edge_bwd0_body(geom_ref, gj_ref, dmg_ref, dvec1_ref, wr1_ref, wr2c_ref,
                    psv_ref, w2at_ref, w2ct_ref, wr1t_ref,
                    dvp_ref, dvn_ref):
    g = geom_ref[...]
    ef = g[:, 0:NB]
    efdr = g[:, NB:2 * NB]
    sh = g[:, 16:19]
    r = g[:, 19:20]
    h = _dot(ef, wr1_ref[...])
    a = _silu(h)
    R2 = _dot(a, wr2c_ref[...])
    sj = gj_ref[...]
    sjv = _dot(sj, psv_ref[...])
    q = R2 * sjv
    dmg = dmg_ref[...]
    dm_s = dmg[:, 0:D]
    dm_v = [dmg[:, D + DV * d:D + DV * (d + 1)] for d in range(3)]
    dR0 = dm_s * sj
    dQ = sum(dm_v[d] * sh[:, d:d + 1] for d in range(3))
    dR2 = dQ * sjv
    dsh = jnp.concatenate(
        [jnp.sum(dm_v[d] * q, axis=1, keepdims=True) for d in range(3)],
        axis=1)
    da = _dot(dR0, w2at_ref[...]) + _dot(dR2, w2ct_ref[...])
    defr = _dot(da * _dsilu(h), wr1t_ref[...])
    u = sh * (1.0 / S3)
    ddot = jnp.sum(defr * efdr, axis=1, keepdims=True)
    udsh = jnp.sum(dsh * u, axis=1, keepdims=True)
    dvec = u * ddot + (S3 / r) * (dsh - u * udsh) + dvec1_ref[...][:, 0:3]
    z = jnp.zeros((EB, 29), jnp.float32)
    dvp_ref[...] = jnp.concatenate([dvec, z], axis=1)
    dvn_ref[...] = jnp.concatenate([-dvec, z], axis=1)


def _segsum_body(b_ref, ne0_ref, en0_ref, en1_ref, out_ref):
    i = pl.program_id(0)

    @pl.when(i == 0)
    def _():
        out_ref[...] = jnp.zeros((8, G), jnp.float32)

    b = b_ref[...][0, 0, :]
    onehot = (b[:, None] == lax.broadcasted_iota(jnp.int32, (NBK, G), 1))
    onehot = onehot.astype(jnp.float32)
    for j, ref in enumerate((ne0_ref, en0_ref, en1_ref)):
        val = ref[...][:, 0:1]
        out_ref[j:j + 1, :] += jnp.sum(val * onehot, axis=0, keepdims=True)


def kernel(positions, node_attrs, shifts, params, edge_index, batch):
    p0, p1 = params['interactions'][0], params['interactions'][1]
    f32 = jnp.float32

    snd = edge_index[0].astype(jnp.int32)
    rcv = edge_index[1].astype(jnp.int32)
    snd_g = jnp.pad(snd, (0, EPAD - E))
    rcv_g = jnp.pad(rcv, (0, EPAD - E))
    snd_s3 = jnp.pad(snd, (0, EPAD - E), constant_values=N).reshape(NS, -1, CH)
    rcv_s3 = jnp.pad(rcv, (0, EPAD - E), constant_values=N).reshape(NS, -1, CH)
    fidx = jnp.concatenate(
        [jnp.pad(snd, (0, EPAD - E), constant_values=N),
         jnp.pad(rcv, (0, EPAD - E), constant_values=N)]).reshape(NS, -1, CH)

    pos_t = jnp.zeros((NPAD, 16), f32).at[:N, 0:3].set(positions)
    na_t = jnp.zeros((NPAD, 16), f32).at[:N, :10].set(node_attrs)
    shifts_t = jnp.zeros((EPAD, 16), f32).at[:E, 0:3].set(shifts)
    batch3 = jnp.pad(batch.astype(jnp.int32), (0, NPAD - N),
                     constant_values=G).reshape(NPAD // NBK, 1, NBK)
    z224 = jnp.zeros((NPAD, 224), f32)
    z32 = jnp.zeros((NPAD, 32), f32)

    def padw(w, rows=16):
        out = jnp.zeros((rows, w.shape[1]), f32)
        return out.at[:w.shape[0]].set(w)

    wread0_8 = jnp.pad(p0['w_read'], ((0, 0), (0, 7)))
    wread2_8 = jnp.pad(p1['wread2'], ((0, 0), (0, 7)))
    ae_8 = jnp.pad(params['atomic_energies'][:, None], ((0, 6), (0, 7)))

    s0, s0up, es0, es1, ne0 = _run_node(
        _node_pre_body, [na_t], [16],
        [padw(params['W_embed']), p0['Wup_s'], padw(p0['w_elem_s']),
         padw(p1['w_elem_s']), ae_8],
        [D, D, D, D, 8])

    psr = _gather(NPAD, 16, 2 * EPAD)(pos_t, jnp.concatenate([snd_g, rcv_g]))
    ps, pr = psr[:EPAD], psr[EPAD:]
    geom, = _run_edge(_geom_body, [ps, pr, shifts_t], [16, 16, 16], [], [32])

    sj0 = _gather(NPAD, D, EPAD)(s0up + psr[0, 0] * 0.0, snd_g)
    msg0, = _run_edge(_edge_fwd0_body, [geom, sj0], [32, D],
                      [p0['Wr1'], p0['Wr2'], p0['P_sv']], [224])
    smsg0 = _scatter_add(EPAD, 224, NPAD)(msg0, rcv_s3, z224)
    s1, v1, gj1t, en0 = _run_node(
        _node0_body, [smsg0, s0, es0], [224, D, D],
        [p0['Wout_s'], p0['Wskip_s'], p0['Wout_v'], wread0_8,
         p1['Wup_s'], p1['Wup_v']],
        [D, 96, 224, 8])

    gj1 = _gather(NPAD, 224, EPAD)(gj1t, snd_g)
    msg1, = _run_edge(_edge_fwd1_body, [geom, gj1], [32, 224],
                      [p1['Wr1'], p1['Wr2'], p1['P_vs'], p1['P_sv']], [224])
    smsg1 = _scatter_add(EPAD, 224, NPAD)(msg1, rcv_s3, z224)
    en1, as1, ds1p = _run_node(
        _node1_body, [smsg1, s1, es1], [224, D, D],
        [p1['Wout_s'], p1['Wskip_s'], p1['Wread1'], wread2_8,
         p1['Wread1'].T, p1['Wout_s'].T, p1['Wskip_s'].T,
         jnp.broadcast_to(p0['w_read'][:, 0][None, :], (1, D))],
        [8, D, D])

    dmg1 = _gather(NPAD, D, EPAD)(as1, rcv_g)
    dgj1, dvec1 = _run_edge(
        _edge_bwd1_body, [geom, gj1, dmg1], [32, 224, D],
        [p1['Wr1'], p1['Wr2'][:, :D + DV], p1['P_vs'].T,
         p1['Wr2'][:, :D].T, p1['Wr2'][:, D:D + DV].T, p1['Wr1'].T],
        [224, 32])
    dsup1 = _scatter_add(EPAD, 224, NPAD)(dgj1, snd_s3, z224)
    a0, = _run_node(
        _node_bwd1_body, [dsup1, ds1p], [224, D],
        [p1['Wup_s'].T, p1['Wup_v'].T, p0['Wout_s'].T, p0['Wout_v'].T],
        [224])

    dmg0 = _gather(NPAD, 224, EPAD)(a0, rcv_g)
    dvp, dvn = _run_edge(
        _edge_bwd0_body, [geom, sj0, dmg0, dvec1], [32, D, 224, 32],
        [p0['Wr1'], p0['Wr2'][:, D + DV:D + 2 * DV], p0['P_sv'],
         p0['Wr2'][:, :D].T, p0['Wr2'][:, D + DV:D + 2 * DV].T, p0['Wr1'].T],
        [32, 32])

    fvals = jnp.concatenate([dvp, dvn], axis=0)
    ftab = _scatter_add(2 * EPAD, 32, NPAD)(fvals, fidx, z32)
    forces = ftab[:N, 0:3]

    esum = pl.pallas_call(
        _segsum_body, grid=(NPAD // NBK,),
        in_specs=[pl.BlockSpec((1, 1, NBK), lambda i: (i, 0, 0)),
                  pl.BlockSpec((NBK, 8), lambda i: (i, 0)),
                  pl.BlockSpec((NBK, 8), lambda i: (i, 0)),
                  pl.BlockSpec((NBK, 8), lambda i: (i, 0))],
        out_specs=pl.BlockSpec((8, G), lambda i: (0, 0)),
        out_shape=jax.ShapeDtypeStruct((8, G), jnp.float32),
    )(batch3, ne0, en0, en1)
    contributions = esum[0:3, :].T
    total = jnp.sum(contributions, axis=-1)
    return total, contributions, forces


# speedup vs baseline: 6.8836x; 1.0003x over previous
import functools

import jax
import jax.numpy as jnp
import numpy as np
from jax import lax
from jax.experimental import pallas as pl
from jax.experimental.pallas import tpu as pltpu
from jax.experimental.pallas import tpu_sc as plsc

N = 10000
E = 320000
D = 128
DV = 32
NB = 8
G = 16
RMAX = 5.0
S3 = float(np.sqrt(3.0))

NPAD = 10240
EPAD = 327680
NC, NS = 2, 16
NW = NC * NS
CH = 128

EB = 2048
NBK = 1024


@functools.lru_cache(maxsize=None)
def _gather(Npad, K, Ep):
    e_w = Ep // NW
    nch = e_w // CH
    mesh = plsc.VectorSubcoreMesh(core_axis_name="c", subcore_axis_name="s")

    NSLOT = 4
    assert nch % NSLOT == 0

    @functools.partial(
        pl.kernel, mesh=mesh,
        out_type=jax.ShapeDtypeStruct((Ep, K), jnp.float32),
        compiler_params=pltpu.CompilerParams(use_tc_tiling_on_sc=False),
        scratch_types=(
            [pltpu.VMEM((e_w,), jnp.int32),
             pltpu.VMEM((NSLOT, CH, K), jnp.float32)]
            + [pltpu.SemaphoreType.DMA] * (2 * NSLOT)
        ),
    )
    def k(table_hbm, idx_hbm, out_hbm, idx_v, buf, *sems):
        gsem, wsem = sems[:NSLOT], sems[NSLOT:]
        wid = lax.axis_index("s") * NC + lax.axis_index("c")
        base = wid * e_w

        def gather(c, b):
            pltpu.async_copy(
                table_hbm.at[idx_v.at[pl.ds(c * CH, CH)]], buf.at[b], gsem[b])

        def wait_gather(c, b):
            pltpu.make_async_copy(
                table_hbm.at[idx_v.at[pl.ds(c * CH, CH)]], buf.at[b], gsem[b]
            ).wait()

        def write(c, b):
            pltpu.async_copy(buf.at[b], out_hbm.at[pl.ds(base + c * CH, CH)],
                             wsem[b])

        def wait_write(c, b):
            pltpu.make_async_copy(
                buf.at[b], out_hbm.at[pl.ds(base + c * CH, CH)], wsem[b]
            ).wait()

        pltpu.sync_copy(idx_hbm.at[pl.ds(base, e_w)], idx_v)
        for b in range(NSLOT):
            gather(b, b)

        def body(g, carry):
            c0 = NSLOT * g
            for b in range(NSLOT):
                wait_gather(c0 + b, b)
                write(c0 + b, b)
            for b in range(NSLOT):
                @pl.when(c0 + NSLOT + b < nch)
                def _(b=b):
                    wait_write(c0 + b, b)
                    gather(c0 + NSLOT + b, b)
            return carry

        lax.fori_loop(0, nch // NSLOT, body, 0)
        for b in range(NSLOT):
            wait_write(nch - NSLOT + b, b)

    return k


@functools.lru_cache(maxsize=None)
def _scatter_add(Ep, K, Npad):
    e_w = Ep // NS
    nch = e_w // CH
    Kh = K // 2
    rows_t = Npad // NS
    mesh = plsc.VectorSubcoreMesh(core_axis_name="c", subcore_axis_name="s")

    assert nch % 2 == 0

    @functools.partial(
        pl.kernel, mesh=mesh,
        out_type=jax.ShapeDtypeStruct((Npad, K), jnp.float32),
        compiler_params=pltpu.CompilerParams(use_tc_tiling_on_sc=False),
        scratch_types=[
            pltpu.VMEM((2, CH), jnp.int32),
            pltpu.VMEM((2, CH, Kh), jnp.float32),
            pltpu.VMEM_SHARED((Npad, Kh), jnp.float32),
            pltpu.SemaphoreType.DMA,
            pltpu.SemaphoreType.DMA,
        ],
    )
    def k(vals_hbm, idx3_hbm, zeros_hbm, out_hbm, idx_c, vbuf, acc,
          semA, semB):
        cid = lax.axis_index("c")
        sid = lax.axis_index("s")
        col0 = cid * Kh
        r0 = sid * rows_t
        pltpu.sync_copy(zeros_hbm.at[pl.ds(r0, rows_t), pl.ds(0, Kh)],
                        acc.at[pl.ds(r0, rows_t)])
        plsc.subcore_barrier()

        def issue(c, slot, sem):
            pltpu.async_copy(idx3_hbm.at[sid, c], idx_c.at[slot], sem)
            pltpu.async_copy(
                vals_hbm.at[pl.ds(sid * e_w + c * CH, CH), pl.ds(col0, Kh)],
                vbuf.at[slot], sem)

        def drain_add(c, slot, sem):
            pltpu.make_async_copy(idx3_hbm.at[sid, c], idx_c.at[slot],
                                  sem).wait()
            pltpu.make_async_copy(
                vals_hbm.at[pl.ds(sid * e_w + c * CH, CH), pl.ds(col0, Kh)],
                vbuf.at[slot], sem).wait()
            pltpu.sync_copy(vbuf.at[slot], acc.at[idx_c.at[slot]], add=True)

        issue(0, 0, semA)

        def body(g, carry):
            c0 = 2 * g
            issue(c0 + 1, 1, semB)
            drain_add(c0, 0, semA)

            @pl.when(c0 + 2 < nch)
            def _():
                issue(c0 + 2, 0, semA)

            drain_add(c0 + 1, 1, semB)
            return carry

        lax.fori_loop(0, nch // 2, body, 0)
        plsc.subcore_barrier()
        pltpu.sync_copy(acc.at[pl.ds(r0, rows_t)],
                        out_hbm.at[pl.ds(r0, rows_t), pl.ds(col0, Kh)])

    return k


def _dot(a, b):
    return jnp.dot(a, b, preferred_element_type=jnp.float32)


def _sigmoid(x):
    return 1.0 / (1.0 + jnp.exp(-x))


def _silu(x):
    return x * _sigmoid(x)


def _dsilu(x):
    s = _sigmoid(x)
    return s * (1.0 + x * (1.0 - s))


def _edge_grid(n_in, n_out):
    eblk = EPAD // EB

    def specs(widths_in, widths_out):
        in_specs = [pl.BlockSpec((EB, w), lambda i: (i, 0)) for w in widths_in]
        out_specs = [pl.BlockSpec((EB, w), lambda i: (i, 0)) for w in widths_out]
        return eblk, in_specs, out_specs

    return specs


def _wspec(shape):
    return pl.BlockSpec(shape, lambda i: tuple(0 for _ in shape))


def _run_edge(body, ins, widths_in, wparams, widths_out):
    eblk = EPAD // EB
    in_specs = ([pl.BlockSpec((EB, w), lambda i: (i, 0)) for w in widths_in]
                + [_wspec(w.shape) for w in wparams])
    out_specs = [pl.BlockSpec((EB, w), lambda i: (i, 0)) for w in widths_out]
    out_shape = [jax.ShapeDtypeStruct((EPAD, w), jnp.float32) for w in widths_out]
    outs = pl.pallas_call(
        body, grid=(eblk,), in_specs=in_specs, out_specs=out_specs,
        out_shape=out_shape,
    )(*ins, *wparams)
    return outs


def _run_node(body, ins, widths_in, wparams, widths_out):
    nblk = NPAD // NBK
    in_specs = ([pl.BlockSpec((NBK, w), lambda i: (i, 0)) for w in widths_in]
                + [_wspec(w.shape) for w in wparams])
    out_specs = [pl.BlockSpec((NBK, w), lambda i: (i, 0)) for w in widths_out]
    out_shape = [jax.ShapeDtypeStruct((NPAD, w), jnp.float32) for w in widths_out]
    outs = pl.pallas_call(
        body, grid=(nblk,), in_specs=in_specs, out_specs=out_specs,
        out_shape=out_shape,
    )(*ins, *wparams)
    return outs


def _node_pre_body(na_ref, wemb_ref, wups0_ref, wes0_ref, wes1_ref, ae_ref,
                   s0_ref, s0up_ref, es0_ref, es1_ref, ne0_ref):
    na = na_ref[...]
    s0 = _dot(na, wemb_ref[...])
    s0_ref[...] = s0
    s0up_ref[...] = _dot(s0, wups0_ref[...])
    es0_ref[...] = _dot(na, wes0_ref[...])
    es1_ref[...] = _dot(na, wes1_ref[...])
    ne0_ref[...] = _dot(na, ae_ref[...])


def _geom_body(ps_ref, pr_ref, sh_ref, geom_ref):
    vec = pr_ref[...][:, 0:3] - ps_ref[...][:, 0:3] + sh_ref[...][:, 0:3]
    r = jnp.sqrt(jnp.sum(vec * vec, axis=1, keepdims=True) + 1e-12)
    u = vec / r
    sh1 = S3 * u
    n = (lax.broadcasted_iota(jnp.int32, (EB, NB), 1) + 1).astype(jnp.float32)
    kn = n * (np.pi / RMAX)
    S = np.sqrt(2.0 / RMAX)
    arg = kn * r
    sn, cs = jnp.sin(arg), jnp.cos(arg)
    b = S * sn / r
    bp = S * (kn * cs / r - sn / (r * r))
    ur = r * (1.0 / RMAX)
    u2 = ur * ur
    u4 = u2 * u2
    u5 = u4 * ur
    env = 1.0 + u5 * (ur * (-28.0 + 48.0 * ur - 21.0 * u2))
    envp = u4 * ur * (-168.0 + 336.0 * ur - 168.0 * u2) * (1.0 / RMAX)
    inside = ur < 1.0
    c = jnp.where(inside, env, 0.0)
    cp = jnp.where(inside, envp, 0.0)
    ef = b * c
    efdr = bp * c + b * cp
    pad = jnp.zeros((EB, 12), jnp.float32)
    geom_ref[...] = jnp.concatenate([ef, efdr, sh1, r, pad], axis=1)


def _edge_fwd0_body(geom_ref, gj_ref, wr1_ref, wr2_ref, psv_ref, msg_ref):
    g = geom_ref[...]
    ef = g[:, 0:NB]
    h = _dot(ef, wr1_ref[...])
    Rm = _dot(_silu(h), wr2_ref[...])
    R0 = Rm[:, 0:D]
    R2 = Rm[:, D + DV:D + 2 * DV]
    sj = gj_ref[...]
    m_s = R0 * sj
    q = R2 * _dot(sj, psv_ref[...])
    mv = [q * g[:, 16 + d:17 + d] for d in range(3)]
    msg_ref[...] = jnp.concatenate([m_s] + mv, axis=1)


def _node0_body(smsg_ref, s0_ref, es0_ref, wouts0_ref, wskips0_ref,
                woutv0_ref, wread0_ref, wups1_ref, wupv1_ref,
                s1_ref, v1_ref, gj1_ref, en0_ref):
    sm = smsg_ref[...]
    s1 = _dot(sm[:, 0:D], wouts0_ref[...]) + _dot(s0_ref[...], wskips0_ref[...]) * es0_ref[...]
    v1 = [_dot(sm[:, D + DV * d:D + DV * (d + 1)], woutv0_ref[...]) for d in range(3)]
    s1_ref[...] = s1
    v1_ref[...] = jnp.concatenate(v1, axis=1)
    en0_ref[...] = _dot(s1, wread0_ref[...])
    s1up = _dot(s1, wups1_ref[...])
    v1up = [_dot(v1[d], wupv1_ref[...]) for d in range(3)]
    gj1_ref[...] = jnp.concatenate([s1up] + v1up, axis=1)


def _edge_fwd1_body(geom_ref, gj_ref, wr1_ref, wr2_ref, pvs_ref, psv_ref,
                    msg_ref):
    g = geom_ref[...]
    ef = g[:, 0:NB]
    h = _dot(ef, wr1_ref[...])
    Rm = _dot(_silu(h), wr2_ref[...])
    R0 = Rm[:, 0:D]
    R1 = Rm[:, D:D + DV]
    R2 = Rm[:, D + DV:D + 2 * DV]
    R3 = Rm[:, D + 2 * DV:D + 3 * DV]
    gj = gj_ref[...]
    sj = gj[:, 0:D]
    vj = [gj[:, D + DV * d:D + DV * (d + 1)] for d in range(3)]
    vdot = sum(vj[d] * g[:, 16 + d:17 + d] for d in range(3))
    m_s = R0 * sj + _dot(R1 * vdot, pvs_ref[...])
    q = R2 * _dot(sj, psv_ref[...])
    mv = [q * g[:, 16 + d:17 + d] + R3 * vj[d] for d in range(3)]
    msg_ref[...] = jnp.concatenate([m_s] + mv, axis=1)


def _node1_body(smsg_ref, s1_ref, es1_ref, wouts1_ref, wskips1_ref,
                wread1_ref, wread2r_ref, wread1t_ref, wouts1t_ref,
                wskips1t_ref, wread0r_ref,
                en1_ref, as1_ref, ds1p_ref):
    sm = smsg_ref[...]
    es1 = es1_ref[...]
    s2 = _dot(sm[:, 0:D], wouts1_ref[...]) + _dot(s1_ref[...], wskips1_ref[...]) * es1
    t = _dot(s2, wread1_ref[...])
    sg = _sigmoid(t)
    en1_ref[...] = _dot(t * sg, wread2r_ref[...])
    gt = (sg * (1.0 + t * (1.0 - sg))) * wread2r_ref[...][:, 0][None, :]
    g_s2 = _dot(gt, wread1t_ref[...])
    as1_ref[...] = _dot(g_s2, wouts1t_ref[...])
    ds1p_ref[...] = _dot(g_s2 * es1, wskips1t_ref[...]) + wread0r_ref[...]


def _edge_bwd1_body(geom_ref, gj_ref, dmg_ref, wr1_ref, wr2ab_ref, pvst_ref,
                    w2at_ref, w2bt_ref, wr1t_ref, dgj_ref, dvec_ref):
    g = geom_ref[...]
    ef = g[:, 0:NB]
    efdr = g[:, NB:2 * NB]
    sh = g[:, 16:19]
    r = g[:, 19:20]
    h = _dot(ef, wr1_ref[...])
    a = _silu(h)
    Rab = _dot(a, wr2ab_ref[...])
    R0 = Rab[:, 0:D]
    R1 = Rab[:, D:D + DV]
    gj = gj_ref[...]
    sj = gj[:, 0:D]
    vj = [gj[:, D + DV * d:D + DV * (d + 1)] for d in range(3)]
    vdot = sum(vj[d] * sh[:, d:d + 1] for d in range(3))
    dm_s = dmg_ref[...]
    dsj = dm_s * R0
    dRV = _dot(dm_s, pvst_ref[...])
    dR1 = dRV * vdot
    dvdot = dRV * R1
    dvj = [dvdot * sh[:, d:d + 1] for d in range(3)]
    dsh = jnp.concatenate(
        [jnp.sum(dvdot * vj[d], axis=1, keepdims=True) for d in range(3)],
        axis=1)
    dR0 = dm_s * sj
    da = _dot(dR0, w2at_ref[...]) + _dot(dR1, w2bt_ref[...])
    defr = _dot(da * _dsilu(h), wr1t_ref[...])
    u = sh * (1.0 / S3)
    ddot = jnp.sum(defr * efdr, axis=1, keepdims=True)
    udsh = jnp.sum(dsh * u, axis=1, keepdims=True)
    dvec = u * ddot + (S3 / r) * (dsh - u * udsh)
    dgj_ref[...] = jnp.concatenate([dsj] + dvj, axis=1)
    dvec_ref[...] = jnp.concatenate(
        [dvec, jnp.zeros((EB, 29), jnp.float32)], axis=1)


def _node_bwd1_body(dsup_ref, ds1p_ref, wups1t_ref, wupv1t_ref, wouts0t_ref,
                    woutv0t_ref, a0_ref):
    dsup = dsup_ref[...]
    ds1 = ds1p_ref[...] + _dot(dsup[:, 0:D], wups1t_ref[...])
    dv1 = [_dot(dsup[:, D + DV * d:D + DV * (d + 1)], wupv1t_ref[...])
           for d in range(3)]
    a_s0 = _dot(ds1, wouts0t_ref[...])
    a_v0 = [_dot(dv1[d], woutv0t_ref[...]) for d in range(3)]
    a0_ref[...] = jnp.concatenate([a_s0] + a_v0, axis=1)


def _edge_bwd0_body(geom_ref, gj_ref, dmg_ref, dvec1_ref, wr1_ref, wr2c_ref,
                    psv_ref, w2at_ref, w2ct_ref, wr1t_ref,
                    dvp_ref, dvn_ref):
    g = geom_ref[...]
    ef = g[:, 0:NB]
    efdr = g[:, NB:2 * NB]
    sh = g[:, 16:19]
    r = g[:, 19:20]
    h = _dot(ef, wr1_ref[...])
    a = _silu(h)
    R2 = _dot(a, wr2c_ref[...])
    sj = gj_ref[...]
    sjv = _dot(sj, psv_ref[...])
    q = R2 * sjv
    dmg = dmg_ref[...]
    dm_s = dmg[:, 0:D]
    dm_v = [dmg[:, D + DV * d:D + DV * (d + 1)] for d in range(3)]
    dR0 = dm_s * sj
    dQ = sum(dm_v[d] * sh[:, d:d + 1] for d in range(3))
    dR2 = dQ * sjv
    dsh = jnp.concatenate(
        [jnp.sum(dm_v[d] * q, axis=1, keepdims=True) for d in range(3)],
        axis=1)
    da = _dot(dR0, w2at_ref[...]) + _dot(dR2, w2ct_ref[...])
    defr = _dot(da * _dsilu(h), wr1t_ref[...])
    u = sh * (1.0 / S3)
    ddot = jnp.sum(defr * efdr, axis=1, keepdims=True)
    udsh = jnp.sum(dsh * u, axis=1, keepdims=True)
    dvec = u * ddot + (S3 / r) * (dsh - u * udsh) + dvec1_ref[...][:, 0:3]
    z = jnp.zeros((EB, 29), jnp.float32)
    dvp_ref[...] = jnp.concatenate([dvec, z], axis=1)
    dvn_ref[...] = jnp.concatenate([-dvec, z], axis=1)


def _segsum_body(b_ref, ne0_ref, en0_ref, en1_ref, out_ref):
    i = pl.program_id(0)

    @pl.when(i == 0)
    def _():
        out_ref[...] = jnp.zeros((8, G), jnp.float32)

    b = b_ref[...][0, 0, :]
    onehot = (b[:, None] == lax.broadcasted_iota(jnp.int32, (NBK, G), 1))
    onehot = onehot.astype(jnp.float32)
    for j, ref in enumerate((ne0_ref, en0_ref, en1_ref)):
        val = ref[...][:, 0:1]
        out_ref[j:j + 1, :] += jnp.sum(val * onehot, axis=0, keepdims=True)


def kernel(positions, node_attrs, shifts, params, edge_index, batch):
    p0, p1 = params['interactions'][0], params['interactions'][1]
    f32 = jnp.float32

    snd = edge_index[0].astype(jnp.int32)
    rcv = edge_index[1].astype(jnp.int32)
    snd_g = jnp.pad(snd, (0, EPAD - E))
    rcv_g = jnp.pad(rcv, (0, EPAD - E))
    snd_s3 = jnp.pad(snd, (0, EPAD - E), constant_values=N).reshape(NS, -1, CH)
    rcv_s3 = jnp.pad(rcv, (0, EPAD - E), constant_values=N).reshape(NS, -1, CH)
    fidx = jnp.concatenate(
        [jnp.pad(snd, (0, EPAD - E), constant_values=N),
         jnp.pad(rcv, (0, EPAD - E), constant_values=N)]).reshape(NS, -1, CH)

    pos_t = jnp.zeros((NPAD, 16), f32).at[:N, 0:3].set(positions)
    na_t = jnp.zeros((NPAD, 16), f32).at[:N, :10].set(node_attrs)
    shifts_t = jnp.zeros((EPAD, 16), f32).at[:E, 0:3].set(shifts)
    batch3 = jnp.pad(batch.astype(jnp.int32), (0, NPAD - N),
                     constant_values=G).reshape(NPAD // NBK, 1, NBK)
    z224 = jnp.zeros((NPAD, 224), f32)
    z32 = jnp.zeros((NPAD, 32), f32)

    def padw(w, rows=16):
        out = jnp.zeros((rows, w.shape[1]), f32)
        return out.at[:w.shape[0]].set(w)

    wread0_8 = jnp.pad(p0['w_read'], ((0, 0), (0, 7)))
    wread2_8 = jnp.pad(p1['wread2'], ((0, 0), (0, 7)))
    ae_8 = jnp.pad(params['atomic_energies'][:, None], ((0, 6), (0, 7)))

    s0, s0up, es0, es1, ne0 = _run_node(
        _node_pre_body, [na_t], [16],
        [padw(params['W_embed']), p0['Wup_s'], padw(p0['w_elem_s']),
         padw(p1['w_elem_s']), ae_8],
        [D, D, D, D, 8])

    psr = _gather(NPAD, 16, 2 * EPAD)(pos_t, jnp.concatenate([snd_g, rcv_g]))
    ps, pr = psr[:EPAD], psr[EPAD:]
    geom, = _run_edge(_geom_body, [ps, pr, shifts_t], [16, 16, 16], [], [32])

    sj0 = _gather(NPAD, D, EPAD)(s0up + psr[0, 0] * 0.0, snd_g)
    msg0, = _run_edge(_edge_fwd0_body, [geom, sj0], [32, D],
                      [p0['Wr1'], p0['Wr2'], p0['P_sv']], [224])
    smsg0 = _scatter_add(EPAD, 224, NPAD)(msg0, rcv_s3, z224)
    s1, v1, gj1t, en0 = _run_node(
        _node0_body, [smsg0, s0, es0], [224, D, D],
        [p0['Wout_s'], p0['Wskip_s'], p0['Wout_v'], wread0_8,
         p1['Wup_s'], p1['Wup_v']],
        [D, 96, 224, 8])

    gj1 = _gather(NPAD, 224, EPAD)(gj1t, snd_g)
    msg1, = _run_edge(_edge_fwd1_body, [geom, gj1], [32, 224],
                      [p1['Wr1'], p1['Wr2'], p1['P_vs'], p1['P_sv']], [224])
    smsg1 = _scatter_add(EPAD, 224, NPAD)(msg1, rcv_s3, z224)
    en1, as1, ds1p = _run_node(
        _node1_body, [smsg1, s1, es1], [224, D, D],
        [p1['Wout_s'], p1['Wskip_s'], p1['Wread1'], wread2_8,
         p1['Wread1'].T, p1['Wout_s'].T, p1['Wskip_s'].T,
         jnp.broadcast_to(p0['w_read'][:, 0][None, :], (1, D))],
        [8, D, D])

    dmg1 = _gather(NPAD, D, EPAD)(as1, rcv_g)
    dgj1, dvec1 = _run_edge(
        _edge_bwd1_body, [geom, gj1, dmg1], [32, 224, D],
        [p1['Wr1'], p1['Wr2'][:, :D + DV], p1['P_vs'].T,
         p1['Wr2'][:, :D].T, p1['Wr2'][:, D:D + DV].T, p1['Wr1'].T],
        [224, 32])
    dsup1 = _scatter_add(EPAD, 224, NPAD)(dgj1, snd_s3, z224)
    a0, = _run_node(
        _node_bwd1_body, [dsup1, ds1p], [224, D],
        [p1['Wup_s'].T, p1['Wup_v'].T, p0['Wout_s'].T, p0['Wout_v'].T],
        [224])

    dmg0 = _gather(NPAD, 224, EPAD)(a0, rcv_g)
    dvp, dvn = _run_edge(
        _edge_bwd0_body, [geom, sj0, dmg0, dvec1], [32, D, 224, 32],
        [p0['Wr1'], p0['Wr2'][:, D + DV:D + 2 * DV], p0['P_sv'],
         p0['Wr2'][:, :D].T, p0['Wr2'][:, D + DV:D + 2 * DV].T, p0['Wr1'].T],
        [32, 32])

    fvals = jnp.concatenate([dvp, dvn], axis=0)
    ftab = _scatter_add(2 * EPAD, 32, NPAD)(fvals, fidx, z32)
    forces = ftab[:N, 0:3]

    esum = pl.pallas_call(
        _segsum_body, grid=(NPAD // NBK,),
        in_specs=[pl.BlockSpec((1, 1, NBK), lambda i: (i, 0, 0)),
                  pl.BlockSpec((NBK, 8), lambda i: (i, 0)),
                  pl.BlockSpec((NBK, 8), lambda i: (i, 0)),
                  pl.BlockSpec((NBK, 8), lambda i: (i, 0))],
        out_specs=pl.BlockSpec((8, G), lambda i: (0, 0)),
        out_shape=jax.ShapeDtypeStruct((8, G), jnp.float32),
    )(batch3, ne0, en0, en1)
    contributions = esum[0:3, :].T
    total = jnp.sum(contributions, axis=-1)
    return total, contributions, forces
